# baseline (speedup 1.0000x reference)
import jax
import jax.numpy as jnp
from jax import lax
from jax.experimental import pallas as pl
from jax.experimental.pallas import tpu as pltpu

_H = 112
_W = 112
_HW = _H * _W
_EPS = 1e-5
_DW_THRESH = 4.0
_PW_THRESH = 0.001


def _fused_kernel(x_ref, w_ref, b1_ref, pw_ref, b2_ref, m_ref, o_ref):
    c_in = x_ref.shape[1]
    x = x_ref[0].reshape(c_in, _HW)

    zcol = jnp.zeros((c_in, 1), jnp.float32)
    xm = m_ref[0:1, :] * jnp.concatenate([zcol, x[:, :-1]], axis=1)
    xp = m_ref[1:2, :] * jnp.concatenate([x[:, 1:], zcol], axis=1)

    def tap(k):
        return w_ref[:, k][:, None]

    p_m1 = tap(0) * xm + tap(1) * x + tap(2) * xp
    p_0 = tap(3) * xm + tap(4) * x + tap(5) * xp
    p_p1 = tap(6) * xm + tap(7) * x + tap(8) * xp

    zrow = jnp.zeros((c_in, _W), jnp.float32)
    y = (b1_ref[...]
         + p_0
         + jnp.concatenate([zrow, p_m1[:, :-_W]], axis=1)
         + jnp.concatenate([p_p1[:, _W:], zrow], axis=1))
    y = jnp.maximum(y, 0.0)

    ymax = jnp.max(y, axis=1, keepdims=True)
    y = jnp.where(ymax < _DW_THRESH, 0.0, y)

    z = jax.lax.dot_general(
        y, pw_ref[...], (((0,), (0,)), ((), ())),
        preferred_element_type=jnp.float32)
    z = jnp.maximum(z + b2_ref[...], 0.0)

    zmax = jnp.max(z, axis=0, keepdims=True)
    z = jnp.where(zmax < _PW_THRESH, 0.0, z)
    o_ref[0] = z


def kernel(x, dw_w, dw_b, bn1_g, bn1_b, bn1_m, bn1_v,
           pw_w, pw_b, bn2_g, bn2_b, bn2_m, bn2_v):
    b, c_in, h, w = x.shape
    c_out = pw_w.shape[0]
    hw = h * w

    inv1 = bn1_g * lax.rsqrt(bn1_v + _EPS)
    w_eff = dw_w.reshape(c_in, 9) * inv1[:, None]
    b1_eff = ((dw_b - bn1_m) * inv1 + bn1_b)[:, None]
    inv2 = bn2_g * lax.rsqrt(bn2_v + _EPS)
    pw_eff = (pw_w * inv2[:, None]).T
    b2_eff = ((pw_b - bn2_m) * inv2 + bn2_b)[None, :]

    wpos = jnp.arange(hw, dtype=jnp.int32) % w
    masks = jnp.stack([(wpos != 0), (wpos != w - 1)]).astype(jnp.float32)

    out = pl.pallas_call(
        _fused_kernel,
        out_shape=jax.ShapeDtypeStruct((b, hw, c_out), jnp.float32),
        grid=(b,),
        in_specs=[
            pl.BlockSpec((1, c_in, h, w), lambda i: (i, 0, 0, 0)),
            pl.BlockSpec((c_in, 9), lambda i: (0, 0)),
            pl.BlockSpec((c_in, 1), lambda i: (0, 0)),
            pl.BlockSpec((c_in, c_out), lambda i: (0, 0)),
            pl.BlockSpec((1, c_out), lambda i: (0, 0)),
            pl.BlockSpec((2, hw), lambda i: (0, 0)),
        ],
        out_specs=pl.BlockSpec((1, hw, c_out), lambda i: (i, 0, 0)),
        compiler_params=pltpu.CompilerParams(
            dimension_semantics=("parallel",),
            vmem_limit_bytes=56 * 1024 * 1024,
        ),
        name="dsconv_fused",
    )(x, w_eff, b1_eff, pw_eff, b2_eff, masks)

    return out.reshape(b, h, w, c_out).transpose(0, 3, 1, 2)


# speedup vs baseline: 1.1084x; 1.1084x over previous
import jax
import jax.numpy as jnp
from jax import lax
from jax.experimental import pallas as pl
from jax.experimental.pallas import tpu as pltpu

_H = 112
_W = 112
_HW = _H * _W
_EPS = 1e-5
_DW_THRESH = 4.0
_PW_THRESH = 0.001


def _fused_kernel(x_ref, w_ref, b1_ref, pw_ref, b2_ref, m_ref, o_ref):
    c_in = x_ref.shape[1]
    x4 = x_ref[0]

    zc = jnp.zeros((c_in, _H, 1), jnp.float32)
    xm = jnp.concatenate([zc, x4[:, :, :-1]], axis=2)
    xq = jnp.concatenate([x4[:, :, 1:], zc], axis=2)

    def tap(k):
        return w_ref[:, k:k + 1, :]

    p_m1 = tap(0) * xm + tap(1) * x4 + tap(2) * xq
    p_0 = tap(3) * xm + tap(4) * x4 + tap(5) * xq
    p_p1 = tap(6) * xm + tap(7) * x4 + tap(8) * xq

    zr = jnp.zeros((c_in, 1, _W), jnp.float32)
    y4 = (p_0
          + jnp.concatenate([zr, p_m1[:, :-1, :]], axis=1)
          + jnp.concatenate([p_p1[:, 1:, :], zr], axis=1))

    y = y4.reshape(c_in, _HW)
    y = jnp.maximum(y + b1_ref[...], 0.0)

    ymax = jnp.max(y, axis=1, keepdims=True)
    y = jnp.where(ymax < _DW_THRESH, 0.0, y)

    z = jax.lax.dot_general(
        y, pw_ref[...], (((0,), (0,)), ((), ())),
        preferred_element_type=jnp.float32)
    z = jnp.maximum(z + b2_ref[...], 0.0)

    zmax = jnp.max(z, axis=0, keepdims=True)
    z = jnp.where(zmax < _PW_THRESH, 0.0, z)
    o_ref[0] = z


def kernel(x, dw_w, dw_b, bn1_g, bn1_b, bn1_m, bn1_v,
           pw_w, pw_b, bn2_g, bn2_b, bn2_m, bn2_v):
    b, c_in, h, w = x.shape
    c_out = pw_w.shape[0]
    hw = h * w

    inv1 = bn1_g * lax.rsqrt(bn1_v + _EPS)
    w_eff = jnp.broadcast_to(
        (dw_w.reshape(c_in, 9) * inv1[:, None])[:, :, None], (c_in, 9, w)
    ) + jnp.zeros((c_in, 9, w), jnp.float32)
    b1_eff = ((dw_b - bn1_m) * inv1 + bn1_b)[:, None]
    inv2 = bn2_g * lax.rsqrt(bn2_v + _EPS)
    pw_eff = (pw_w * inv2[:, None]).T
    b2_eff = ((pw_b - bn2_m) * inv2 + bn2_b)[None, :]

    wpos = jnp.arange(hw, dtype=jnp.int32) % w
    masks = jnp.stack([(wpos != 0), (wpos != w - 1)]).astype(jnp.float32)

    out = pl.pallas_call(
        _fused_kernel,
        out_shape=jax.ShapeDtypeStruct((b, hw, c_out), jnp.float32),
        grid=(b,),
        in_specs=[
            pl.BlockSpec((1, c_in, h, w), lambda i: (i, 0, 0, 0)),
            pl.BlockSpec((c_in, 9, w), lambda i: (0, 0, 0)),
            pl.BlockSpec((c_in, 1), lambda i: (0, 0)),
            pl.BlockSpec((c_in, c_out), lambda i: (0, 0)),
            pl.BlockSpec((1, c_out), lambda i: (0, 0)),
            pl.BlockSpec((2, hw), lambda i: (0, 0)),
        ],
        out_specs=pl.BlockSpec((1, hw, c_out), lambda i: (i, 0, 0)),
        compiler_params=pltpu.CompilerParams(
            dimension_semantics=("parallel",),
            vmem_limit_bytes=56 * 1024 * 1024,
        ),
        name="dsconv_fused",
    )(x, w_eff, b1_eff, pw_eff, b2_eff, masks)

    return out.reshape(b, h, w, c_out).transpose(0, 3, 1, 2)


# speedup vs baseline: 1.1135x; 1.0046x over previous
import jax
import jax.numpy as jnp
from jax import lax
from jax.experimental import pallas as pl
from jax.experimental.pallas import tpu as pltpu

_H = 112
_W = 112
_HW = _H * _W
_EPS = 1e-5
_DW_THRESH = 4.0
_PW_THRESH = 0.001


def _fused_kernel(x_ref, w_ref, b1_ref, pw_ref, b2_ref, o_ref):
    c_in = x_ref.shape[1]
    x4 = x_ref[0]

    zc = jnp.zeros((c_in, _H, 1), jnp.float32)
    xm = jnp.concatenate([zc, x4[:, :, :-1]], axis=2)
    xq = jnp.concatenate([x4[:, :, 1:], zc], axis=2)

    def tap(k):
        return w_ref[:, k:k + 1, :]

    p_m1 = tap(0) * xm + tap(1) * x4 + tap(2) * xq
    p_0 = tap(3) * xm + tap(4) * x4 + tap(5) * xq
    p_p1 = tap(6) * xm + tap(7) * x4 + tap(8) * xq

    zr = jnp.zeros((c_in, 1, _W), jnp.float32)
    y4 = (p_0
          + jnp.concatenate([zr, p_m1[:, :-1, :]], axis=1)
          + jnp.concatenate([p_p1[:, 1:, :], zr], axis=1))

    y = y4.reshape(c_in, _HW)
    y = jnp.maximum(y + b1_ref[...], 0.0)

    ymax = jnp.max(y, axis=1, keepdims=True)
    y = jnp.where(ymax < _DW_THRESH, 0.0, y)

    z = jax.lax.dot_general(
        y, pw_ref[...], (((0,), (0,)), ((), ())),
        preferred_element_type=jnp.float32)
    z = jnp.maximum(z + b2_ref[...], 0.0)

    zmax = jnp.max(z, axis=0, keepdims=True)
    z = jnp.where(zmax < _PW_THRESH, 0.0, z)
    o_ref[0] = z


def kernel(x, dw_w, dw_b, bn1_g, bn1_b, bn1_m, bn1_v,
           pw_w, pw_b, bn2_g, bn2_b, bn2_m, bn2_v):
    b, c_in, h, w = x.shape
    c_out = pw_w.shape[0]
    hw = h * w

    inv1 = bn1_g * lax.rsqrt(bn1_v + _EPS)
    w_eff = jnp.broadcast_to(
        (dw_w.reshape(c_in, 9) * inv1[:, None])[:, :, None], (c_in, 9, w)
    ) + jnp.zeros((c_in, 9, w), jnp.float32)
    b1_eff = ((dw_b - bn1_m) * inv1 + bn1_b)[:, None]
    inv2 = bn2_g * lax.rsqrt(bn2_v + _EPS)
    pw_eff = (pw_w * inv2[:, None]).T
    b2_eff = ((pw_b - bn2_m) * inv2 + bn2_b)[None, :]

    out = pl.pallas_call(
        _fused_kernel,
        out_shape=jax.ShapeDtypeStruct((b, hw, c_out), jnp.float32),
        grid=(b,),
        in_specs=[
            pl.BlockSpec((1, c_in, h, w), lambda i: (i, 0, 0, 0)),
            pl.BlockSpec((c_in, 9, w), lambda i: (0, 0, 0)),
            pl.BlockSpec((c_in, 1), lambda i: (0, 0)),
            pl.BlockSpec((c_in, c_out), lambda i: (0, 0)),
            pl.BlockSpec((1, c_out), lambda i: (0, 0)),
        ],
        out_specs=pl.BlockSpec((1, hw, c_out), lambda i: (i, 0, 0)),
        compiler_params=pltpu.CompilerParams(
            dimension_semantics=("parallel",),
            vmem_limit_bytes=56 * 1024 * 1024,
        ),
        name="dsconv_fused",
    )(x, w_eff, b1_eff, pw_eff, b2_eff)

    return out.reshape(b, h, w, c_out).transpose(0, 3, 1, 2)
